# baseline (speedup 1.0000x reference)
import functools
import math

import jax
import jax.numpy as jnp
from jax import lax
from jax.experimental import pallas as pl
from jax.experimental.pallas import tpu as pltpu
from jax.experimental.pallas import tpu_sc as plsc

N = 10000
NPAD = 10240
G = 128
H = 128
E = 320000
EPAD = 327680
NW = 32
EW = EPAD // NW
C = 128
NCH = EW // C
C3 = 64
NCH3 = EW // C3
NSL = NPAD // 16
NRW = NPAD // NW


def _dg(x, idx):
    return jnp.take_along_axis(x, idx, axis=0)


def _proj_body(x_ref, w_ref, b_ref, q_ref, k_ref, v_ref, s_ref):
    y = (
        jnp.dot(x_ref[...], w_ref[...], preferred_element_type=jnp.float32)
        + b_ref[...]
    )
    q_ref[...] = y[:, 0:128]
    k_ref[...] = y[:, 128:256]
    v_ref[...] = y[:, 256:384]
    s_ref[...] = y[:, 384:512]


def _projections(x, W_all, b_all):
    bn = 1024
    grid = NPAD // bn
    o = jax.ShapeDtypeStruct((NPAD, H), jnp.float32)
    return pl.pallas_call(
        _proj_body,
        grid=(grid,),
        in_specs=[
            pl.BlockSpec((bn, G), lambda i: (i, 0)),
            pl.BlockSpec((G, 4 * H), lambda i: (0, 0)),
            pl.BlockSpec((1, 4 * H), lambda i: (0, 0)),
        ],
        out_specs=[pl.BlockSpec((bn, H), lambda i: (i, 0))] * 4,
        out_shape=[o, o, o, o],
    )(x, W_all, b_all)


def _k2_body(q_hbm, k_hbm, dst_hbm, src_hbm, alpha_hbm, maxpart_hbm,
             dst_bufs, src_bufs, q_bufs, k_bufs, alpha_buf, bins, tr_buf,
             sems):
    cid = lax.axis_index("c")
    sid = lax.axis_index("s")
    wid = sid * 2 + cid
    neginf = jnp.full((16,), -jnp.inf, jnp.float32)
    iota = lax.iota(jnp.int32, 16)
    iota16 = iota * 16

    def init_body(i, carry):
        bins[pl.ds(i * 16, 16)] = neginf
        return carry

    lax.fori_loop(0, NPAD // 16, init_body, 0)

    def prefetch(ci, p):
        base = wid * EW + ci * C
        pltpu.sync_copy(dst_hbm.at[pl.ds(base, C)], dst_bufs[p])
        pltpu.sync_copy(src_hbm.at[pl.ds(base, C)], src_bufs[p])
        pltpu.async_copy(q_hbm.at[dst_bufs[p]], q_bufs[p], sems[p])
        pltpu.async_copy(k_hbm.at[src_bufs[p]], k_bufs[p], sems[p])

    def wait(p):
        pltpu.make_async_copy(q_hbm.at[dst_bufs[p]], q_bufs[p], sems[p]).wait()
        pltpu.make_async_copy(k_hbm.at[src_bufs[p]], k_bufs[p], sems[p]).wait()

    def compute(ci, p):
        base = wid * EW + ci * C
        q_buf, k_buf, dst_buf = q_bufs[p], k_bufs[p], dst_bufs[p]

        def group(g, carry):
            for e in range(16):
                row = g * 16 + e
                acc = q_buf[row, pl.ds(0, 16)] * k_buf[row, pl.ds(0, 16)]
                for b in range(1, 8):
                    acc = acc + (
                        q_buf[row, pl.ds(b * 16, 16)]
                        * k_buf[row, pl.ds(b * 16, 16)]
                    )
                tr_buf[pl.ds(e * 16, 16)] = acc
            alpha16 = plsc.load_gather(tr_buf, [iota16])
            for j in range(1, 16):
                alpha16 = alpha16 + plsc.load_gather(tr_buf, [iota16 + j])
            alpha_buf[pl.ds(g * 16, 16)] = alpha16
            dst16 = dst_buf[pl.ds(g * 16, 16)]
            sk, sv = plsc.sort_key_val(dst16, alpha16)
            for sh in (1, 2, 4, 8):
                idxs = jnp.maximum(iota - sh, 0)
                pk = _dg(sk, idxs)
                pv = _dg(sv, idxs)
                valid = (iota >= sh) & (pk == sk)
                sv = jnp.where(valid, jnp.maximum(sv, pv), sv)
            nk = _dg(sk, jnp.minimum(iota + 1, 15))
            is_last = (iota == 15) | (sk != nk)
            cur = plsc.load_gather(bins, [sk])
            plsc.store_scatter(
                bins, [sk], jnp.maximum(cur, sv), mask=is_last
            )
            return carry

        lax.fori_loop(0, 8, group, 0)
        pltpu.sync_copy(alpha_buf, alpha_hbm.at[pl.ds(base, C)])

    prefetch(0, 0)

    def chunk_body(ci2, carry):
        wait(0)
        prefetch(2 * ci2 + 1, 1)
        compute(2 * ci2, 0)
        wait(1)
        prefetch(2 * ci2 + 2, 0)
        compute(2 * ci2 + 1, 1)
        return carry

    lax.fori_loop(0, NCH // 2 - 1, chunk_body, 0)
    wait(0)
    prefetch(NCH - 1, 1)
    compute(NCH - 2, 0)
    wait(1)
    compute(NCH - 1, 1)
    pltpu.sync_copy(bins, maxpart_hbm.at[pl.ds(wid * NPAD, NPAD)])


def _k2(q, k, dstp, srcp):
    mesh = plsc.VectorSubcoreMesh(core_axis_name="c", subcore_axis_name="s")
    fn = pl.kernel(
        _k2_body,
        compiler_params=pltpu.CompilerParams(needs_layout_passes=False),
        out_type=[
            jax.ShapeDtypeStruct((EPAD,), jnp.float32),
            jax.ShapeDtypeStruct((NW * NPAD,), jnp.float32),
        ],
        mesh=mesh,
        scratch_types=[
            [pltpu.VMEM((C,), jnp.int32)] * 2,
            [pltpu.VMEM((C,), jnp.int32)] * 2,
            [pltpu.VMEM((C, H), jnp.float32)] * 2,
            [pltpu.VMEM((C, H), jnp.float32)] * 2,
            pltpu.VMEM((C,), jnp.float32),
            pltpu.VMEM((NPAD,), jnp.float32),
            pltpu.VMEM((256,), jnp.float32),
            [pltpu.SemaphoreType.DMA] * 2,
        ],
    )
    return fn(q, k, dstp, srcp)


def _k3_body(v_hbm, dst_hbm, src_hbm, alpha_hbm, maxpart_hbm,
             outpart_hbm, denpart_hbm, segmax_hbm,
             dst_bufs, src_bufs, idx_buf, v_bufs, alpha_bufs, w_bufs, smax,
             mp_buf, acc_sp, den_sp, sems):
    cid = lax.axis_index("c")
    sid = lax.axis_index("s")
    wid = sid * 2 + cid
    zero16 = jnp.zeros((16,), jnp.float32)
    iota = lax.iota(jnp.int32, 16)
    v_buf, w_buf = v_bufs[0], w_bufs[0]

    def fill_idx(lo):
        for t in range(C3 // 16):
            idx_buf[pl.ds(t * 16, 16)] = iota + (lo + t * 16)

    for i in range(C3):
        for b in range(8):
            v_buf[i, pl.ds(b * 16, 16)] = zero16
    for t in range(C3 // 16):
        w_buf[pl.ds(t * 16, 16)] = zero16
    for r in range(NSL // C3):
        fill_idx(sid * NSL + r * C3)
        pltpu.sync_copy(v_buf, acc_sp.at[idx_buf])
        pltpu.sync_copy(w_buf, den_sp.at[idx_buf])

    pltpu.sync_copy(
        maxpart_hbm.at[pl.ds(sid * NSL, NSL)],
        smax.at[pl.ds(sid * NSL, NSL)],
    )
    for j in range(1, NW):
        pltpu.sync_copy(
            maxpart_hbm.at[pl.ds(j * NPAD + sid * NSL, NSL)], mp_buf
        )

        def mx(t, carry, _j=j):
            m = jnp.maximum(
                mp_buf[pl.ds(t * 16, 16)],
                smax[pl.ds(sid * NSL + t * 16, 16)],
            )
            smax[pl.ds(sid * NSL + t * 16, 16)] = m
            return carry

        lax.fori_loop(0, NSL // 16, mx, 0)
    pltpu.sync_copy(
        smax.at[pl.ds(sid * NSL, NSL)], segmax_hbm.at[pl.ds(sid * NSL, NSL)]
    )
    plsc.subcore_barrier()
    pltpu.sync_copy(segmax_hbm, smax)

    def prefetch(ci, p):
        base = wid * EW + ci * C3
        pltpu.sync_copy(dst_hbm.at[pl.ds(base, C3)], dst_bufs[p])
        pltpu.sync_copy(src_hbm.at[pl.ds(base, C3)], src_bufs[p])
        pltpu.async_copy(v_hbm.at[src_bufs[p]], v_bufs[p], sems[p])
        pltpu.async_copy(
            alpha_hbm.at[pl.ds(base, C3)], alpha_bufs[p], sems[p]
        )

    def wait(p):
        pltpu.make_async_copy(
            v_hbm.at[src_bufs[p]], v_bufs[p], sems[p]
        ).wait()
        pltpu.make_async_copy(
            alpha_hbm.at[pl.ds(0, C3)], alpha_bufs[p], sems[p]
        ).wait()

    def compute(p):
        vb, wb, ab, db = v_bufs[p], w_bufs[p], alpha_bufs[p], dst_bufs[p]

        def group(g, carry):
            a16 = ab[pl.ds(g * 16, 16)]
            dst16 = db[pl.ds(g * 16, 16)]
            m16 = plsc.load_gather(smax, [dst16])
            w16 = jnp.exp(a16 - m16)
            wb[pl.ds(g * 16, 16)] = w16
            for e in range(16):
                row = g * 16 + e
                b = _dg(w16, jnp.full((16,), e, jnp.int32))
                for bb in range(8):
                    vb[row, pl.ds(bb * 16, 16)] = (
                        vb[row, pl.ds(bb * 16, 16)] * b
                    )
            return carry

        lax.fori_loop(0, C3 // 16, group, 0)
        pltpu.sync_copy(vb, acc_sp.at[db], add=True)
        pltpu.sync_copy(wb, den_sp.at[db], add=True)

    prefetch(0, 0)

    def chunk_body(ci2, carry):
        wait(0)
        prefetch(2 * ci2 + 1, 1)
        compute(0)
        wait(1)
        prefetch(2 * ci2 + 2, 0)
        compute(1)
        return carry

    lax.fori_loop(0, NCH3 // 2 - 1, chunk_body, 0)
    wait(0)
    prefetch(NCH3 - 1, 1)
    compute(0)
    wait(1)
    compute(1)

    plsc.subcore_barrier()
    for r in range(NSL // C3):
        lo = sid * NSL + r * C3
        fill_idx(lo)
        pltpu.sync_copy(acc_sp.at[idx_buf], v_buf)
        pltpu.sync_copy(v_buf, outpart_hbm.at[cid, pl.ds(lo, C3)])
        pltpu.sync_copy(den_sp.at[idx_buf], w_buf)
        pltpu.sync_copy(w_buf, denpart_hbm.at[pl.ds(cid * NPAD + lo, C3)])


def _k3(v, dstp, srcp, alpha, maxpart):
    mesh = plsc.VectorSubcoreMesh(core_axis_name="c", subcore_axis_name="s")
    fn = pl.kernel(
        _k3_body,
        compiler_params=pltpu.CompilerParams(needs_layout_passes=False),
        out_type=[
            jax.ShapeDtypeStruct((2, NPAD, H), jnp.float32),
            jax.ShapeDtypeStruct((2 * NPAD,), jnp.float32),
            jax.ShapeDtypeStruct((NPAD,), jnp.float32),
        ],
        mesh=mesh,
        scratch_types=[
            [pltpu.VMEM((C3,), jnp.int32)] * 2,
            [pltpu.VMEM((C3,), jnp.int32)] * 2,
            pltpu.VMEM((C3,), jnp.int32),
            [pltpu.VMEM((C3, H), jnp.float32)] * 2,
            [pltpu.VMEM((C3,), jnp.float32)] * 2,
            [pltpu.VMEM((C3,), jnp.float32)] * 2,
            pltpu.VMEM((NPAD,), jnp.float32),
            pltpu.VMEM((NSL,), jnp.float32),
            pltpu.VMEM_SHARED((NPAD, H), jnp.float32),
            pltpu.VMEM_SHARED((NPAD,), jnp.float32),
            [pltpu.SemaphoreType.DMA] * 2,
        ],
    )
    out, den, _ = fn(v, dstp, srcp, alpha, maxpart)
    return out, den


def _k3b_body(outpart_hbm, denpart_hbm, skip_hbm, final_hbm,
              p0_buf, p1_buf, s_buf, d0_buf, d1_buf):
    cid = lax.axis_index("c")
    sid = lax.axis_index("s")
    wid = sid * 2 + cid
    iota = lax.iota(jnp.int32, 16)
    eps = jnp.full((16,), 1e-16, jnp.float32)
    one = jnp.full((16,), 1.0, jnp.float32)

    def blk(r, carry):
        lo = wid * NRW + r * C3
        pltpu.sync_copy(outpart_hbm.at[0, pl.ds(lo, C3)], p0_buf)
        pltpu.sync_copy(outpart_hbm.at[1, pl.ds(lo, C3)], p1_buf)
        pltpu.sync_copy(skip_hbm.at[pl.ds(lo, C3)], s_buf)
        pltpu.sync_copy(denpart_hbm.at[pl.ds(lo, C3)], d0_buf)
        pltpu.sync_copy(denpart_hbm.at[pl.ds(NPAD + lo, C3)], d1_buf)
        for g in range(C3 // 16):
            den = d0_buf[pl.ds(g * 16, 16)] + d1_buf[pl.ds(g * 16, 16)]
            recip = one / (den + eps)
            for e in range(16):
                row = g * 16 + e
                b = _dg(recip, jnp.full((16,), e, jnp.int32))
                for bb in range(8):
                    sl = pl.ds(bb * 16, 16)
                    p0_buf[row, sl] = (
                        (p0_buf[row, sl] + p1_buf[row, sl]) * b
                        + s_buf[row, sl]
                    )
        pltpu.sync_copy(p0_buf, final_hbm.at[pl.ds(lo, C3)])
        return carry

    lax.fori_loop(0, NRW // C3, blk, 0)


def _k3b(outpart, denpart, skip):
    mesh = plsc.VectorSubcoreMesh(core_axis_name="c", subcore_axis_name="s")
    fn = pl.kernel(
        _k3b_body,
        compiler_params=pltpu.CompilerParams(needs_layout_passes=False),
        out_type=jax.ShapeDtypeStruct((NPAD, H), jnp.float32),
        mesh=mesh,
        scratch_types=[
            pltpu.VMEM((C3, H), jnp.float32),
            pltpu.VMEM((C3, H), jnp.float32),
            pltpu.VMEM((C3, H), jnp.float32),
            pltpu.VMEM((C3,), jnp.float32),
            pltpu.VMEM((C3,), jnp.float32),
        ],
    )
    return fn(outpart, denpart, skip)


def kernel(node_features, edge_index, edge_norm, edge_type,
           Wq, bq, Wk, bk, Wv, bv, Ws, bs):
    d = Wq.shape[0]
    scale = 1.0 / math.sqrt(d)
    W_all = jnp.concatenate([Wq.T * scale, Wk.T, Wv.T, Ws.T], axis=1)
    b_all = jnp.concatenate([bq * scale, bk, bv, bs])[None, :]
    xpad = jnp.pad(node_features, ((0, NPAD - N), (0, 0)))
    q, k, v, skip = _projections(xpad, W_all, b_all)

    npad_e = EPAD - E
    pad_ids = (N + (jnp.arange(npad_e, dtype=jnp.int32) % (NPAD - N))).astype(
        jnp.int32
    )
    srcp = jnp.concatenate([edge_index[0], pad_ids])
    dstp = jnp.concatenate([edge_index[1], pad_ids])

    alpha, maxpart = _k2(q, k, dstp, srcp)
    outpart, denpart = _k3(v, dstp, srcp, alpha, maxpart)
    out = _k3b(outpart, denpart, skip)
    return out[:N]


# speedup vs baseline: 1.0277x; 1.0277x over previous
import functools
import math

import jax
import jax.numpy as jnp
from jax import lax
from jax.experimental import pallas as pl
from jax.experimental.pallas import tpu as pltpu
from jax.experimental.pallas import tpu_sc as plsc

N = 10000
NPAD = 10240
G = 128
H = 128
E = 320000
EPAD = 327680
NW = 32
EW = EPAD // NW
C = 128
NCH = EW // C
C3 = 128
NCH3 = EW // C3
CS = 64
CB = 64
NSL = NPAD // 16
NRW = NPAD // NW


def _dg(x, idx):
    return jnp.take_along_axis(x, idx, axis=0)


def _proj_body(x_ref, w_ref, b_ref, q_ref, k_ref, v_ref, s_ref):
    y = (
        jnp.dot(x_ref[...], w_ref[...], preferred_element_type=jnp.float32)
        + b_ref[...]
    )
    q_ref[...] = y[:, 0:128]
    k_ref[...] = y[:, 128:256]
    v_ref[...] = y[:, 256:384]
    s_ref[...] = y[:, 384:512]


def _projections(x, W_all, b_all):
    bn = 1024
    grid = NPAD // bn
    o = jax.ShapeDtypeStruct((NPAD, H), jnp.float32)
    return pl.pallas_call(
        _proj_body,
        grid=(grid,),
        in_specs=[
            pl.BlockSpec((bn, G), lambda i: (i, 0)),
            pl.BlockSpec((G, 4 * H), lambda i: (0, 0)),
            pl.BlockSpec((1, 4 * H), lambda i: (0, 0)),
        ],
        out_specs=[pl.BlockSpec((bn, H), lambda i: (i, 0))] * 4,
        out_shape=[o, o, o, o],
    )(x, W_all, b_all)


def _k2_body(q_hbm, k_hbm, dst_hbm, src_hbm, alpha_hbm, maxpart_hbm,
             dst_bufs, src_bufs, q_bufs, k_bufs, alpha_buf, bins, tr_buf,
             sems):
    cid = lax.axis_index("c")
    sid = lax.axis_index("s")
    wid = sid * 2 + cid
    neginf = jnp.full((16,), -jnp.inf, jnp.float32)
    iota = lax.iota(jnp.int32, 16)
    iota16 = iota * 16

    def init_body(i, carry):
        bins[pl.ds(i * 16, 16)] = neginf
        return carry

    lax.fori_loop(0, NPAD // 16, init_body, 0)

    def prefetch(ci, p):
        base = wid * EW + ci * C
        pltpu.sync_copy(dst_hbm.at[pl.ds(base, C)], dst_bufs[p])
        pltpu.sync_copy(src_hbm.at[pl.ds(base, C)], src_bufs[p])
        pltpu.async_copy(q_hbm.at[dst_bufs[p]], q_bufs[p], sems[p])
        pltpu.async_copy(k_hbm.at[src_bufs[p]], k_bufs[p], sems[p])

    def wait(p):
        pltpu.make_async_copy(q_hbm.at[dst_bufs[p]], q_bufs[p], sems[p]).wait()
        pltpu.make_async_copy(k_hbm.at[src_bufs[p]], k_bufs[p], sems[p]).wait()

    def compute(ci, p):
        base = wid * EW + ci * C
        q_buf, k_buf, dst_buf = q_bufs[p], k_bufs[p], dst_bufs[p]

        def group(g, carry):
            for e in range(16):
                row = g * 16 + e
                acc = q_buf[row, pl.ds(0, 16)] * k_buf[row, pl.ds(0, 16)]
                for b in range(1, 8):
                    acc = acc + (
                        q_buf[row, pl.ds(b * 16, 16)]
                        * k_buf[row, pl.ds(b * 16, 16)]
                    )
                tr_buf[pl.ds(e * 16, 16)] = acc
            alpha16 = plsc.load_gather(tr_buf, [iota16])
            for j in range(1, 16):
                alpha16 = alpha16 + plsc.load_gather(tr_buf, [iota16 + j])
            alpha_buf[pl.ds(g * 16, 16)] = alpha16
            dst16 = dst_buf[pl.ds(g * 16, 16)]
            sk, sv = plsc.sort_key_val(dst16, alpha16)
            for sh in (1, 2, 4, 8):
                idxs = jnp.maximum(iota - sh, 0)
                pk = _dg(sk, idxs)
                pv = _dg(sv, idxs)
                valid = (iota >= sh) & (pk == sk)
                sv = jnp.where(valid, jnp.maximum(sv, pv), sv)
            nk = _dg(sk, jnp.minimum(iota + 1, 15))
            is_last = (iota == 15) | (sk != nk)
            cur = plsc.load_gather(bins, [sk])
            plsc.store_scatter(
                bins, [sk], jnp.maximum(cur, sv), mask=is_last
            )
            return carry

        lax.fori_loop(0, 8, group, 0)
        pltpu.sync_copy(alpha_buf, alpha_hbm.at[pl.ds(base, C)])

    prefetch(0, 0)

    def chunk_body(ci2, carry):
        wait(0)
        prefetch(2 * ci2 + 1, 1)
        compute(2 * ci2, 0)
        wait(1)
        prefetch(2 * ci2 + 2, 0)
        compute(2 * ci2 + 1, 1)
        return carry

    lax.fori_loop(0, NCH // 2 - 1, chunk_body, 0)
    wait(0)
    prefetch(NCH - 1, 1)
    compute(NCH - 2, 0)
    wait(1)
    compute(NCH - 1, 1)
    pltpu.sync_copy(bins, maxpart_hbm.at[pl.ds(wid * NPAD, NPAD)])


def _k2(q, k, dstp, srcp):
    mesh = plsc.VectorSubcoreMesh(core_axis_name="c", subcore_axis_name="s")
    fn = pl.kernel(
        _k2_body,
        compiler_params=pltpu.CompilerParams(needs_layout_passes=False),
        out_type=[
            jax.ShapeDtypeStruct((EPAD,), jnp.float32),
            jax.ShapeDtypeStruct((NW * NPAD,), jnp.float32),
        ],
        mesh=mesh,
        scratch_types=[
            [pltpu.VMEM((C,), jnp.int32)] * 2,
            [pltpu.VMEM((C,), jnp.int32)] * 2,
            [pltpu.VMEM((C, H), jnp.float32)] * 2,
            [pltpu.VMEM((C, H), jnp.float32)] * 2,
            pltpu.VMEM((C,), jnp.float32),
            pltpu.VMEM((NPAD,), jnp.float32),
            pltpu.VMEM((256,), jnp.float32),
            [pltpu.SemaphoreType.DMA] * 2,
        ],
    )
    return fn(q, k, dstp, srcp)


def _k3_body(v_hbm, dst_hbm, src_hbm, alpha_hbm, maxpart_hbm,
             outpart_hbm, denpart_hbm, segmax_hbm,
             dst_bufs, dlo_bufs, dhi_bufs, src_bufs, idx_buf, v_bufs,
             alpha_bufs, w_bufs, smax, mp_buf, acc_sp, den_sp, sems):
    cid = lax.axis_index("c")
    sid = lax.axis_index("s")
    wid = sid * 2 + cid
    zero16 = jnp.zeros((16,), jnp.float32)
    iota = lax.iota(jnp.int32, 16)
    v_buf, w_buf = v_bufs[0], w_bufs[0]

    def fill_idx(lo):
        for t in range(CS // 16):
            idx_buf[pl.ds(t * 16, 16)] = iota + (lo + t * 16)

    for i in range(CS):
        for b in range(8):
            v_buf[i, pl.ds(b * 16, 16)] = zero16
    for t in range(CS // 16):
        w_buf[pl.ds(t * 16, 16)] = zero16
    for r in range(NSL // CS):
        fill_idx(sid * NSL + r * CS)
        pltpu.sync_copy(v_buf.at[pl.ds(0, CS)], acc_sp.at[idx_buf])
        pltpu.sync_copy(w_buf.at[pl.ds(0, CS)], den_sp.at[idx_buf])

    pltpu.sync_copy(
        maxpart_hbm.at[pl.ds(sid * NSL, NSL)],
        smax.at[pl.ds(sid * NSL, NSL)],
    )
    for j in range(1, NW):
        pltpu.sync_copy(
            maxpart_hbm.at[pl.ds(j * NPAD + sid * NSL, NSL)], mp_buf
        )

        def mx(t, carry, _j=j):
            m = jnp.maximum(
                mp_buf[pl.ds(t * 16, 16)],
                smax[pl.ds(sid * NSL + t * 16, 16)],
            )
            smax[pl.ds(sid * NSL + t * 16, 16)] = m
            return carry

        lax.fori_loop(0, NSL // 16, mx, 0)
    pltpu.sync_copy(
        smax.at[pl.ds(sid * NSL, NSL)], segmax_hbm.at[pl.ds(sid * NSL, NSL)]
    )
    plsc.subcore_barrier()
    pltpu.sync_copy(segmax_hbm, smax)

    def prefetch(ci, p):
        base = wid * EW + ci * C3
        pltpu.sync_copy(dst_hbm.at[pl.ds(base, C3)], dst_bufs[p])
        pltpu.sync_copy(dst_hbm.at[pl.ds(base, CS)], dlo_bufs[p])
        pltpu.sync_copy(dst_hbm.at[pl.ds(base + CS, CS)], dhi_bufs[p])
        pltpu.sync_copy(src_hbm.at[pl.ds(base, C3)], src_bufs[p])
        pltpu.async_copy(v_hbm.at[src_bufs[p]], v_bufs[p], sems[p])
        pltpu.async_copy(
            alpha_hbm.at[pl.ds(base, C3)], alpha_bufs[p], sems[p]
        )

    def wait(p):
        pltpu.make_async_copy(
            v_hbm.at[src_bufs[p]], v_bufs[p], sems[p]
        ).wait()
        pltpu.make_async_copy(
            alpha_hbm.at[pl.ds(0, C3)], alpha_bufs[p], sems[p]
        ).wait()

    def compute(p):
        vb, wb, ab, db = v_bufs[p], w_bufs[p], alpha_bufs[p], dst_bufs[p]

        def group(g, carry):
            a16 = ab[pl.ds(g * 16, 16)]
            dst16 = db[pl.ds(g * 16, 16)]
            m16 = plsc.load_gather(smax, [dst16])
            w16 = jnp.exp(a16 - m16)
            wb[pl.ds(g * 16, 16)] = w16
            for e in range(16):
                row = g * 16 + e
                b = _dg(w16, jnp.full((16,), e, jnp.int32))
                for bb in range(8):
                    vb[row, pl.ds(bb * 16, 16)] = (
                        vb[row, pl.ds(bb * 16, 16)] * b
                    )
            return carry

        lax.fori_loop(0, C3 // 16, group, 0)
        pltpu.sync_copy(vb.at[pl.ds(0, CS)], acc_sp.at[dlo_bufs[p]], add=True)
        pltpu.sync_copy(vb.at[pl.ds(CS, CS)], acc_sp.at[dhi_bufs[p]], add=True)
        pltpu.sync_copy(wb.at[pl.ds(0, CS)], den_sp.at[dlo_bufs[p]], add=True)
        pltpu.sync_copy(wb.at[pl.ds(CS, CS)], den_sp.at[dhi_bufs[p]], add=True)

    prefetch(0, 0)

    def chunk_body(ci2, carry):
        wait(0)
        prefetch(2 * ci2 + 1, 1)
        compute(0)
        wait(1)
        prefetch(2 * ci2 + 2, 0)
        compute(1)
        return carry

    lax.fori_loop(0, NCH3 // 2 - 1, chunk_body, 0)
    wait(0)
    prefetch(NCH3 - 1, 1)
    compute(0)
    wait(1)
    compute(1)

    plsc.subcore_barrier()
    for r in range(NSL // CS):
        lo = sid * NSL + r * CS
        fill_idx(lo)
        pltpu.sync_copy(acc_sp.at[idx_buf], v_buf.at[pl.ds(0, CS)])
        pltpu.sync_copy(
            v_buf.at[pl.ds(0, CS)], outpart_hbm.at[cid, pl.ds(lo, CS)]
        )
        pltpu.sync_copy(den_sp.at[idx_buf], w_buf.at[pl.ds(0, CS)])
        pltpu.sync_copy(
            w_buf.at[pl.ds(0, CS)],
            denpart_hbm.at[pl.ds(cid * NPAD + lo, CS)],
        )


def _k3(v, dstp, srcp, alpha, maxpart):
    mesh = plsc.VectorSubcoreMesh(core_axis_name="c", subcore_axis_name="s")
    fn = pl.kernel(
        _k3_body,
        compiler_params=pltpu.CompilerParams(needs_layout_passes=False),
        out_type=[
            jax.ShapeDtypeStruct((2, NPAD, H), jnp.float32),
            jax.ShapeDtypeStruct((2 * NPAD,), jnp.float32),
            jax.ShapeDtypeStruct((NPAD,), jnp.float32),
        ],
        mesh=mesh,
        scratch_types=[
            [pltpu.VMEM((C3,), jnp.int32)] * 2,
            [pltpu.VMEM((CS,), jnp.int32)] * 2,
            [pltpu.VMEM((CS,), jnp.int32)] * 2,
            [pltpu.VMEM((C3,), jnp.int32)] * 2,
            pltpu.VMEM((CS,), jnp.int32),
            [pltpu.VMEM((C3, H), jnp.float32)] * 2,
            [pltpu.VMEM((C3,), jnp.float32)] * 2,
            [pltpu.VMEM((C3,), jnp.float32)] * 2,
            pltpu.VMEM((NPAD,), jnp.float32),
            pltpu.VMEM((NSL,), jnp.float32),
            pltpu.VMEM_SHARED((NPAD, H), jnp.float32),
            pltpu.VMEM_SHARED((NPAD,), jnp.float32),
            [pltpu.SemaphoreType.DMA] * 2,
        ],
    )
    out, den, _ = fn(v, dstp, srcp, alpha, maxpart)
    return out, den


def _k3b_body(outpart_hbm, denpart_hbm, skip_hbm, final_hbm,
              p0_buf, p1_buf, s_buf, d0_buf, d1_buf):
    cid = lax.axis_index("c")
    sid = lax.axis_index("s")
    wid = sid * 2 + cid
    iota = lax.iota(jnp.int32, 16)
    eps = jnp.full((16,), 1e-16, jnp.float32)
    one = jnp.full((16,), 1.0, jnp.float32)

    def blk(r, carry):
        lo = wid * NRW + r * CB
        pltpu.sync_copy(outpart_hbm.at[0, pl.ds(lo, CB)], p0_buf)
        pltpu.sync_copy(outpart_hbm.at[1, pl.ds(lo, CB)], p1_buf)
        pltpu.sync_copy(skip_hbm.at[pl.ds(lo, CB)], s_buf)
        pltpu.sync_copy(denpart_hbm.at[pl.ds(lo, CB)], d0_buf)
        pltpu.sync_copy(denpart_hbm.at[pl.ds(NPAD + lo, CB)], d1_buf)
        for g in range(CB // 16):
            den = d0_buf[pl.ds(g * 16, 16)] + d1_buf[pl.ds(g * 16, 16)]
            recip = one / (den + eps)
            for e in range(16):
                row = g * 16 + e
                b = _dg(recip, jnp.full((16,), e, jnp.int32))
                for bb in range(8):
                    sl = pl.ds(bb * 16, 16)
                    p0_buf[row, sl] = (
                        (p0_buf[row, sl] + p1_buf[row, sl]) * b
                        + s_buf[row, sl]
                    )
        pltpu.sync_copy(p0_buf, final_hbm.at[pl.ds(lo, CB)])
        return carry

    lax.fori_loop(0, NRW // CB, blk, 0)


def _k3b(outpart, denpart, skip):
    mesh = plsc.VectorSubcoreMesh(core_axis_name="c", subcore_axis_name="s")
    fn = pl.kernel(
        _k3b_body,
        compiler_params=pltpu.CompilerParams(needs_layout_passes=False),
        out_type=jax.ShapeDtypeStruct((NPAD, H), jnp.float32),
        mesh=mesh,
        scratch_types=[
            pltpu.VMEM((CB, H), jnp.float32),
            pltpu.VMEM((CB, H), jnp.float32),
            pltpu.VMEM((CB, H), jnp.float32),
            pltpu.VMEM((CB,), jnp.float32),
            pltpu.VMEM((CB,), jnp.float32),
        ],
    )
    return fn(outpart, denpart, skip)


def kernel(node_features, edge_index, edge_norm, edge_type,
           Wq, bq, Wk, bk, Wv, bv, Ws, bs):
    d = Wq.shape[0]
    scale = 1.0 / math.sqrt(d)
    W_all = jnp.concatenate([Wq.T * scale, Wk.T, Wv.T, Ws.T], axis=1)
    b_all = jnp.concatenate([bq * scale, bk, bv, bs])[None, :]
    xpad = jnp.pad(node_features, ((0, NPAD - N), (0, 0)))
    q, k, v, skip = _projections(xpad, W_all, b_all)

    npad_e = EPAD - E
    pad_ids = (N + (jnp.arange(npad_e, dtype=jnp.int32) % (NPAD - N))).astype(
        jnp.int32
    )
    srcp = jnp.concatenate([edge_index[0], pad_ids])
    dstp = jnp.concatenate([edge_index[1], pad_ids])

    alpha, maxpart = _k2(q, k, dstp, srcp)
    outpart, denpart = _k3(v, dstp, srcp, alpha, maxpart)
    out = _k3b(outpart, denpart, skip)
    return out[:N]


# speedup vs baseline: 1.3602x; 1.3236x over previous
import functools
import math

import jax
import jax.numpy as jnp
from jax import lax
from jax.experimental import pallas as pl
from jax.experimental.pallas import tpu as pltpu
from jax.experimental.pallas import tpu_sc as plsc

N = 10000
NPAD = 10240
G = 128
H = 128
E = 320000
EPAD = 327680
NW = 32
EW = EPAD // NW
C = 128
NCH = EW // C
C3 = 128
NCH3 = EW // C3
CS = 64
CB = 64
NSL = NPAD // 16
NRW = NPAD // NW


def _dg(x, idx):
    return jnp.take_along_axis(x, idx, axis=0)


def _proj_body(x_ref, w_ref, b_ref, q_ref, k_ref, v_ref, s_ref):
    y = (
        jnp.dot(x_ref[...], w_ref[...], preferred_element_type=jnp.float32)
        + b_ref[...]
    )
    q_ref[...] = y[:, 0:128]
    k_ref[...] = y[:, 128:256]
    v_ref[...] = y[:, 256:384]
    s_ref[...] = y[:, 384:512]


def _projections(x, W_all, b_all):
    bn = 1024
    grid = NPAD // bn
    o = jax.ShapeDtypeStruct((NPAD, H), jnp.float32)
    return pl.pallas_call(
        _proj_body,
        grid=(grid,),
        in_specs=[
            pl.BlockSpec((bn, G), lambda i: (i, 0)),
            pl.BlockSpec((G, 4 * H), lambda i: (0, 0)),
            pl.BlockSpec((1, 4 * H), lambda i: (0, 0)),
        ],
        out_specs=[pl.BlockSpec((bn, H), lambda i: (i, 0))] * 4,
        out_shape=[o, o, o, o],
    )(x, W_all, b_all)


def _k2_body(q_hbm, k_hbm, dst_hbm, src_hbm, alpha_hbm, maxpart_hbm,
             dst_all, src_all, q_bufs, k_bufs, alpha_buf, bins, tr_buf,
             sems):
    cid = lax.axis_index("c")
    sid = lax.axis_index("s")
    wid = sid * 2 + cid
    neginf = jnp.full((16,), -jnp.inf, jnp.float32)
    iota = lax.iota(jnp.int32, 16)
    iota16 = iota * 16

    pltpu.sync_copy(dst_hbm.at[pl.ds(wid * EW, EW)], dst_all)
    pltpu.sync_copy(src_hbm.at[pl.ds(wid * EW, EW)], src_all)

    def init_body(i, carry):
        bins[pl.ds(i * 16, 16)] = neginf
        return carry

    lax.fori_loop(0, NPAD // 16, init_body, 0)

    def prefetch(ci, p):
        pltpu.async_copy(
            q_hbm.at[dst_all.at[pl.ds(ci * C, C)]], q_bufs[p], sems[p]
        )
        pltpu.async_copy(
            k_hbm.at[src_all.at[pl.ds(ci * C, C)]], k_bufs[p], sems[p]
        )

    def wait(p):
        pltpu.make_async_copy(
            q_hbm.at[dst_all.at[pl.ds(0, C)]], q_bufs[p], sems[p]
        ).wait()
        pltpu.make_async_copy(
            k_hbm.at[src_all.at[pl.ds(0, C)]], k_bufs[p], sems[p]
        ).wait()

    def compute(ci, p):
        base = wid * EW + ci * C
        q_buf, k_buf = q_bufs[p], k_bufs[p]

        def group(g, carry):
            for e in range(16):
                row = g * 16 + e
                acc = q_buf[row, pl.ds(0, 16)] * k_buf[row, pl.ds(0, 16)]
                for b in range(1, 8):
                    acc = acc + (
                        q_buf[row, pl.ds(b * 16, 16)]
                        * k_buf[row, pl.ds(b * 16, 16)]
                    )
                tr_buf[pl.ds(e * 16, 16)] = acc
            alpha16 = plsc.load_gather(tr_buf, [iota16])
            for j in range(1, 16):
                alpha16 = alpha16 + plsc.load_gather(tr_buf, [iota16 + j])
            alpha_buf[pl.ds(g * 16, 16)] = alpha16
            dst16 = dst_all[pl.ds(ci * C + g * 16, 16)]
            sk, sv = plsc.sort_key_val(dst16, alpha16)
            for sh in (1, 2, 4, 8):
                idxs = jnp.maximum(iota - sh, 0)
                pk = _dg(sk, idxs)
                pv = _dg(sv, idxs)
                valid = (iota >= sh) & (pk == sk)
                sv = jnp.where(valid, jnp.maximum(sv, pv), sv)
            nk = _dg(sk, jnp.minimum(iota + 1, 15))
            is_last = (iota == 15) | (sk != nk)
            cur = plsc.load_gather(bins, [sk])
            plsc.store_scatter(
                bins, [sk], jnp.maximum(cur, sv), mask=is_last
            )
            return carry

        lax.fori_loop(0, 8, group, 0)
        pltpu.sync_copy(alpha_buf, alpha_hbm.at[pl.ds(base, C)])

    prefetch(0, 0)

    def chunk_body(ci2, carry):
        wait(0)
        prefetch(2 * ci2 + 1, 1)
        compute(2 * ci2, 0)
        wait(1)
        prefetch(2 * ci2 + 2, 0)
        compute(2 * ci2 + 1, 1)
        return carry

    lax.fori_loop(0, NCH // 2 - 1, chunk_body, 0)
    wait(0)
    prefetch(NCH - 1, 1)
    compute(NCH - 2, 0)
    wait(1)
    compute(NCH - 1, 1)
    pltpu.sync_copy(bins, maxpart_hbm.at[pl.ds(wid * NPAD, NPAD)])


def _k2(q, k, dstp, srcp):
    mesh = plsc.VectorSubcoreMesh(core_axis_name="c", subcore_axis_name="s")
    fn = pl.kernel(
        _k2_body,
        compiler_params=pltpu.CompilerParams(needs_layout_passes=False),
        out_type=[
            jax.ShapeDtypeStruct((EPAD,), jnp.float32),
            jax.ShapeDtypeStruct((NW * NPAD,), jnp.float32),
        ],
        mesh=mesh,
        scratch_types=[
            pltpu.VMEM((EW,), jnp.int32),
            pltpu.VMEM((EW,), jnp.int32),
            [pltpu.VMEM((C, H), jnp.float32)] * 2,
            [pltpu.VMEM((C, H), jnp.float32)] * 2,
            pltpu.VMEM((C,), jnp.float32),
            pltpu.VMEM((NPAD,), jnp.float32),
            pltpu.VMEM((256,), jnp.float32),
            [pltpu.SemaphoreType.DMA] * 2,
        ],
    )
    return fn(q, k, dstp, srcp)


def _k3_body(v_hbm, dst_hbm, src_hbm, alpha_hbm, maxpart_hbm,
             outpart_hbm, denpart_hbm, segmax_hbm,
             dst_bufs, dlo_bufs, dhi_bufs, src_bufs, idx_buf, v_bufs,
             alpha_bufs, w_bufs, smax, mp_buf, acc_sp, den_sp, sems):
    cid = lax.axis_index("c")
    sid = lax.axis_index("s")
    wid = sid * 2 + cid
    zero16 = jnp.zeros((16,), jnp.float32)
    iota = lax.iota(jnp.int32, 16)
    v_buf, w_buf = v_bufs[0], w_bufs[0]

    def fill_idx(lo):
        for t in range(CS // 16):
            idx_buf[pl.ds(t * 16, 16)] = iota + (lo + t * 16)

    for i in range(CS):
        for b in range(8):
            v_buf[i, pl.ds(b * 16, 16)] = zero16
    for t in range(CS // 16):
        w_buf[pl.ds(t * 16, 16)] = zero16
    for r in range(NSL // CS):
        fill_idx(sid * NSL + r * CS)
        pltpu.sync_copy(v_buf.at[pl.ds(0, CS)], acc_sp.at[idx_buf])
        pltpu.sync_copy(w_buf.at[pl.ds(0, CS)], den_sp.at[idx_buf])

    pltpu.sync_copy(
        maxpart_hbm.at[pl.ds(sid * NSL, NSL)],
        smax.at[pl.ds(sid * NSL, NSL)],
    )
    for j in range(1, NW):
        pltpu.sync_copy(
            maxpart_hbm.at[pl.ds(j * NPAD + sid * NSL, NSL)], mp_buf
        )

        def mx(t, carry, _j=j):
            m = jnp.maximum(
                mp_buf[pl.ds(t * 16, 16)],
                smax[pl.ds(sid * NSL + t * 16, 16)],
            )
            smax[pl.ds(sid * NSL + t * 16, 16)] = m
            return carry

        lax.fori_loop(0, NSL // 16, mx, 0)
    pltpu.sync_copy(
        smax.at[pl.ds(sid * NSL, NSL)], segmax_hbm.at[pl.ds(sid * NSL, NSL)]
    )
    plsc.subcore_barrier()
    pltpu.sync_copy(segmax_hbm, smax)

    def prefetch(ci, p):
        base = wid * EW + ci * C3
        pltpu.sync_copy(src_hbm.at[pl.ds(base, C3)], src_bufs[p])
        pltpu.async_copy(v_hbm.at[src_bufs[p]], v_bufs[p], sems[p])
        pltpu.async_copy(dst_hbm.at[pl.ds(base, C3)], dst_bufs[p], sems[p])
        pltpu.async_copy(dst_hbm.at[pl.ds(base, CS)], dlo_bufs[p], sems[p])
        pltpu.async_copy(
            dst_hbm.at[pl.ds(base + CS, CS)], dhi_bufs[p], sems[p]
        )
        pltpu.async_copy(
            alpha_hbm.at[pl.ds(base, C3)], alpha_bufs[p], sems[p]
        )

    def wait(p):
        pltpu.make_async_copy(
            v_hbm.at[src_bufs[p]], v_bufs[p], sems[p]
        ).wait()
        pltpu.make_async_copy(
            dst_hbm.at[pl.ds(0, C3)], dst_bufs[p], sems[p]
        ).wait()
        pltpu.make_async_copy(
            dst_hbm.at[pl.ds(0, CS)], dlo_bufs[p], sems[p]
        ).wait()
        pltpu.make_async_copy(
            dst_hbm.at[pl.ds(0, CS)], dhi_bufs[p], sems[p]
        ).wait()
        pltpu.make_async_copy(
            alpha_hbm.at[pl.ds(0, C3)], alpha_bufs[p], sems[p]
        ).wait()

    def compute(p):
        vb, wb, ab, db = v_bufs[p], w_bufs[p], alpha_bufs[p], dst_bufs[p]

        def group(g, carry):
            a16 = ab[pl.ds(g * 16, 16)]
            dst16 = db[pl.ds(g * 16, 16)]
            m16 = plsc.load_gather(smax, [dst16])
            w16 = jnp.exp(a16 - m16)
            wb[pl.ds(g * 16, 16)] = w16
            for e in range(16):
                row = g * 16 + e
                b = _dg(w16, jnp.full((16,), e, jnp.int32))
                for bb in range(8):
                    vb[row, pl.ds(bb * 16, 16)] = (
                        vb[row, pl.ds(bb * 16, 16)] * b
                    )
            return carry

        lax.fori_loop(0, C3 // 16, group, 0)
        pltpu.sync_copy(vb.at[pl.ds(0, CS)], acc_sp.at[dlo_bufs[p]], add=True)
        pltpu.sync_copy(vb.at[pl.ds(CS, CS)], acc_sp.at[dhi_bufs[p]], add=True)
        pltpu.sync_copy(wb.at[pl.ds(0, CS)], den_sp.at[dlo_bufs[p]], add=True)
        pltpu.sync_copy(wb.at[pl.ds(CS, CS)], den_sp.at[dhi_bufs[p]], add=True)

    prefetch(0, 0)

    def chunk_body(ci2, carry):
        wait(0)
        prefetch(2 * ci2 + 1, 1)
        compute(0)
        wait(1)
        prefetch(2 * ci2 + 2, 0)
        compute(1)
        return carry

    lax.fori_loop(0, NCH3 // 2 - 1, chunk_body, 0)
    wait(0)
    prefetch(NCH3 - 1, 1)
    compute(0)
    wait(1)
    compute(1)

    plsc.subcore_barrier()
    for r in range(NSL // CS):
        lo = sid * NSL + r * CS
        fill_idx(lo)
        pltpu.sync_copy(acc_sp.at[idx_buf], v_buf.at[pl.ds(0, CS)])
        pltpu.sync_copy(
            v_buf.at[pl.ds(0, CS)], outpart_hbm.at[cid, pl.ds(lo, CS)]
        )
        pltpu.sync_copy(den_sp.at[idx_buf], w_buf.at[pl.ds(0, CS)])
        pltpu.sync_copy(
            w_buf.at[pl.ds(0, CS)],
            denpart_hbm.at[pl.ds(cid * NPAD + lo, CS)],
        )


def _k3(v, dstp, srcp, alpha, maxpart):
    mesh = plsc.VectorSubcoreMesh(core_axis_name="c", subcore_axis_name="s")
    fn = pl.kernel(
        _k3_body,
        compiler_params=pltpu.CompilerParams(needs_layout_passes=False),
        out_type=[
            jax.ShapeDtypeStruct((2, NPAD, H), jnp.float32),
            jax.ShapeDtypeStruct((2 * NPAD,), jnp.float32),
            jax.ShapeDtypeStruct((NPAD,), jnp.float32),
        ],
        mesh=mesh,
        scratch_types=[
            [pltpu.VMEM((C3,), jnp.int32)] * 2,
            [pltpu.VMEM((CS,), jnp.int32)] * 2,
            [pltpu.VMEM((CS,), jnp.int32)] * 2,
            [pltpu.VMEM((C3,), jnp.int32)] * 2,
            pltpu.VMEM((CS,), jnp.int32),
            [pltpu.VMEM((C3, H), jnp.float32)] * 2,
            [pltpu.VMEM((C3,), jnp.float32)] * 2,
            [pltpu.VMEM((C3,), jnp.float32)] * 2,
            pltpu.VMEM((NPAD,), jnp.float32),
            pltpu.VMEM((NSL,), jnp.float32),
            pltpu.VMEM_SHARED((NPAD, H), jnp.float32),
            pltpu.VMEM_SHARED((NPAD,), jnp.float32),
            [pltpu.SemaphoreType.DMA] * 2,
        ],
    )
    out, den, _ = fn(v, dstp, srcp, alpha, maxpart)
    return out, den


def _k3b_body(outpart_hbm, denpart_hbm, skip_hbm, final_hbm,
              p0_buf, p1_buf, s_buf, d0_buf, d1_buf):
    cid = lax.axis_index("c")
    sid = lax.axis_index("s")
    wid = sid * 2 + cid
    iota = lax.iota(jnp.int32, 16)
    eps = jnp.full((16,), 1e-16, jnp.float32)
    one = jnp.full((16,), 1.0, jnp.float32)

    def blk(r, carry):
        lo = wid * NRW + r * CB
        pltpu.sync_copy(outpart_hbm.at[0, pl.ds(lo, CB)], p0_buf)
        pltpu.sync_copy(outpart_hbm.at[1, pl.ds(lo, CB)], p1_buf)
        pltpu.sync_copy(skip_hbm.at[pl.ds(lo, CB)], s_buf)
        pltpu.sync_copy(denpart_hbm.at[pl.ds(lo, CB)], d0_buf)
        pltpu.sync_copy(denpart_hbm.at[pl.ds(NPAD + lo, CB)], d1_buf)
        for g in range(CB // 16):
            den = d0_buf[pl.ds(g * 16, 16)] + d1_buf[pl.ds(g * 16, 16)]
            recip = one / (den + eps)
            for e in range(16):
                row = g * 16 + e
                b = _dg(recip, jnp.full((16,), e, jnp.int32))
                for bb in range(8):
                    sl = pl.ds(bb * 16, 16)
                    p0_buf[row, sl] = (
                        (p0_buf[row, sl] + p1_buf[row, sl]) * b
                        + s_buf[row, sl]
                    )
        pltpu.sync_copy(p0_buf, final_hbm.at[pl.ds(lo, CB)])
        return carry

    lax.fori_loop(0, NRW // CB, blk, 0)


def _k3b(outpart, denpart, skip):
    mesh = plsc.VectorSubcoreMesh(core_axis_name="c", subcore_axis_name="s")
    fn = pl.kernel(
        _k3b_body,
        compiler_params=pltpu.CompilerParams(needs_layout_passes=False),
        out_type=jax.ShapeDtypeStruct((NPAD, H), jnp.float32),
        mesh=mesh,
        scratch_types=[
            pltpu.VMEM((CB, H), jnp.float32),
            pltpu.VMEM((CB, H), jnp.float32),
            pltpu.VMEM((CB, H), jnp.float32),
            pltpu.VMEM((CB,), jnp.float32),
            pltpu.VMEM((CB,), jnp.float32),
        ],
    )
    return fn(outpart, denpart, skip)


def kernel(node_features, edge_index, edge_norm, edge_type,
           Wq, bq, Wk, bk, Wv, bv, Ws, bs):
    d = Wq.shape[0]
    scale = 1.0 / math.sqrt(d)
    W_all = jnp.concatenate([Wq.T * scale, Wk.T, Wv.T, Ws.T], axis=1)
    b_all = jnp.concatenate([bq * scale, bk, bv, bs])[None, :]
    xpad = jnp.pad(node_features, ((0, NPAD - N), (0, 0)))
    q, k, v, skip = _projections(xpad, W_all, b_all)

    npad_e = EPAD - E
    pad_ids = (N + (jnp.arange(npad_e, dtype=jnp.int32) % (NPAD - N))).astype(
        jnp.int32
    )
    srcp = jnp.concatenate([edge_index[0], pad_ids])
    dstp = jnp.concatenate([edge_index[1], pad_ids])

    alpha, maxpart = _k2(q, k, dstp, srcp)
    outpart, denpart = _k3(v, dstp, srcp, alpha, maxpart)
    out = _k3b(outpart, denpart, skip)
    return out[:N]
